# baseline (speedup 1.0000x reference)
import functools

import jax
import jax.numpy as jnp
from jax import lax
from jax.experimental import pallas as pl
from jax.experimental.pallas import tpu as pltpu
from jax.experimental.pallas import tpu_sc as plsc

_NUM_STYLES = 64
_DIM = 192
_BATCH = 16384
_LANES = 16


def _table_body(emb_ref, w_ref, b_ref, p_ref):
    p_ref[...] = (
        lax.dot_general(
            emb_ref[...],
            w_ref[...],
            (((1,), (1,)), ((), ())),
            preferred_element_type=jnp.float32,
        )
        + b_ref[...]
    )


def _make_sc_kernel():
    info = plsc.get_sparse_core_info()
    nc, ns = info.num_cores, info.num_subcores
    nw = nc * ns
    bpw = _BATCH // nw
    nch = 4
    ch = bpw // nch
    nvec = _DIM // _LANES

    mesh = plsc.VectorSubcoreMesh(core_axis_name="c", subcore_axis_name="s")

    @functools.partial(
        pl.kernel,
        mesh=mesh,
        compiler_params=pltpu.CompilerParams(
            needs_layout_passes=False, use_tc_tiling_on_sc=False
        ),
        out_type=jax.ShapeDtypeStruct((nw, nch, ch, _DIM), jnp.float32),
        scratch_types=[
            pltpu.VMEM((nch, ch), jnp.int32),
            pltpu.VMEM((nch, ch, _DIM), jnp.float32),
            pltpu.VMEM((bpw,), jnp.float32),
            pltpu.SemaphoreType.DMA,
            pltpu.SemaphoreType.DMA,
        ],
    )
    def sc_kernel(sid_hbm, exa_hbm, p_hbm, out_hbm, idx_v, rows_v, exa_v, gsem, osem):
        wid = lax.axis_index("s") * nc + lax.axis_index("c")
        pltpu.sync_copy(sid_hbm.at[wid], idx_v)
        gathers = [
            pltpu.async_copy(p_hbm.at[idx_v.at[k]], rows_v.at[k], gsem)
            for k in range(nch)
        ]
        pltpu.sync_copy(exa_hbm.at[wid], exa_v)

        stores = []
        for k in range(nch):
            gathers[k].wait()

            def body(r, _):
                g = k * ch + r
                e = plsc.load_gather(exa_v, [jnp.full((_LANES,), g, jnp.int32)])
                for j in range(nvec):
                    sl = pl.ds(j * _LANES, _LANES)
                    rows_v[k, r, sl] = rows_v[k, r, sl] * e
                return _

            lax.fori_loop(0, ch, body, 0, unroll=2)
            stores.append(pltpu.async_copy(rows_v.at[k], out_hbm.at[wid, k], osem))
        for st in stores:
            st.wait()

    return sc_kernel, nw, nch, ch


_SC_KERNEL, _NW, _NCH, _CH = _make_sc_kernel()


def kernel(style_id, exaggeration, emb, W, b):
    p = pl.pallas_call(
        _table_body,
        out_shape=jax.ShapeDtypeStruct((_NUM_STYLES, _DIM), jnp.float32),
    )(emb, W, b.reshape(1, _DIM))
    sid = style_id.reshape(_NW, _NCH, _CH)
    exa = exaggeration.reshape(_NW, _NCH * _CH)
    out = _SC_KERNEL(sid, exa, p)
    return out.reshape(_BATCH, _DIM)
